# baseline (speedup 1.0000x reference)
import functools

import jax
import jax.numpy as jnp
from jax import lax
from jax.experimental import pallas as pl
from jax.experimental.pallas import tpu as pltpu
from jax.experimental.pallas import tpu_sc as plsc

EMBED_DIM = 64
SCALE = float(EMBED_DIM) ** 0.5

NUM_CORES = 2
NUM_SUBCORES = 16
NW = NUM_CORES * NUM_SUBCORES

CHUNK = 128


def _body(nchunk, rows_per_w, ids_hbm, table_hbm, out_hbm,
          idx_v, rows0, rows1, sem0, sem1):
  wid = lax.axis_index("s") * NUM_CORES + lax.axis_index("c")
  base = wid * rows_per_w

  pltpu.sync_copy(ids_hbm.at[wid], idx_v)

  def gather(c, buf, sem):
    return pltpu.async_copy(table_hbm.at[idx_v.at[c]], buf, sem)

  def scale_store(c, buf):
    def row_body(r, carry):
      for cc in range(EMBED_DIM // 16):
        sl = pl.ds(cc * 16, 16)
        buf[r, sl] = buf[r, sl] * SCALE
      return carry
    lax.fori_loop(0, CHUNK, row_body, 0)
    pltpu.sync_copy(buf, out_hbm.at[pl.ds(base + c * CHUNK, CHUNK)])

  gather(0, rows0, sem0)

  def loop_body(g, carry):
    c = 2 * g
    gather(c + 1, rows1, sem1)
    pltpu.make_async_copy(table_hbm.at[idx_v.at[c]], rows0, sem0).wait()
    scale_store(c, rows0)

    @pl.when(g < (nchunk // 2) - 1)
    def _():
      gather(c + 2, rows0, sem0)

    pltpu.make_async_copy(table_hbm.at[idx_v.at[c + 1]], rows1, sem1).wait()
    scale_store(c + 1, rows1)
    return carry

  lax.fori_loop(0, nchunk // 2, loop_body, 0)


@jax.jit
def kernel(input_ids, table):
  batch, seq_len = input_ids.shape
  vocab, d = table.shape
  n = batch * seq_len
  rows_per_w = n // NW
  nchunk = rows_per_w // CHUNK

  ids3 = input_ids.reshape(NW, nchunk, CHUNK).astype(jnp.int32)

  mesh = plsc.VectorSubcoreMesh(core_axis_name="c", subcore_axis_name="s")
  fn = pl.kernel(
      functools.partial(_body, nchunk, rows_per_w),
      out_type=jax.ShapeDtypeStruct((n, d), jnp.float32),
      mesh=mesh,
      compiler_params=pltpu.CompilerParams(use_tc_tiling_on_sc=False),
      scratch_types=[
          pltpu.VMEM((nchunk, CHUNK), jnp.int32),
          pltpu.VMEM((CHUNK, d), jnp.float32),
          pltpu.VMEM((CHUNK, d), jnp.float32),
          pltpu.SemaphoreType.DMA,
          pltpu.SemaphoreType.DMA,
      ],
  )
  out = fn(ids3, table)
  return out.reshape(batch, seq_len, d)


# speedup vs baseline: 1.0590x; 1.0590x over previous
import functools

import jax
import jax.numpy as jnp
from jax import lax
from jax.experimental import pallas as pl
from jax.experimental.pallas import tpu as pltpu
from jax.experimental.pallas import tpu_sc as plsc

EMBED_DIM = 64
SCALE = float(EMBED_DIM) ** 0.5

NUM_CORES = 2
NUM_SUBCORES = 16
NW = NUM_CORES * NUM_SUBCORES

CHUNK = 128
NBUF = 4


def _body(nchunk, rows_per_w, ids_hbm, table_hbm, out_hbm,
          idx_v, bufs_g, bufs_s, sems_g, sems_s):
  wid = lax.axis_index("s") * NUM_CORES + lax.axis_index("c")
  base = wid * rows_per_w
  nouter = nchunk // NBUF

  pltpu.sync_copy(ids_hbm.at[wid], idx_v)

  def gather(c, b):
    return pltpu.async_copy(table_hbm.at[idx_v.at[c]], bufs_g[b], sems_g[b])

  def gather_wait(c, b):
    pltpu.make_async_copy(table_hbm.at[idx_v.at[c]], bufs_g[b],
                          sems_g[b]).wait()

  def store(c, b):
    return pltpu.async_copy(
        bufs_s[b], out_hbm.at[pl.ds(base + c * CHUNK, CHUNK)], sems_s[b])

  def store_wait(c, b):
    pltpu.make_async_copy(
        bufs_s[b], out_hbm.at[pl.ds(base + c * CHUNK, CHUNK)],
        sems_s[b]).wait()

  for b in range(NBUF):
    gather(b, b)

  def loop_body(t, carry):
    for b in range(NBUF):
      c = t * NBUF + b
      gather_wait(c, b)

      @pl.when(t > 0)
      def _():
        store_wait(c - NBUF, b)

      @plsc.parallel_loop(0, CHUNK, unroll=4)
      def _(r):
        for cc in range(EMBED_DIM // 16):
          sl = pl.ds(cc * 16, 16)
          bufs_s[b][r, sl] = bufs_g[b][r, sl] * SCALE

      store(c, b)

      @pl.when(t < nouter - 1)
      def _():
        gather(c + NBUF, b)
    return carry

  lax.fori_loop(0, nouter, loop_body, 0)

  for b in range(NBUF):
    store_wait((nouter - 1) * NBUF + b, b)


@jax.jit
def kernel(input_ids, table):
  batch, seq_len = input_ids.shape
  vocab, d = table.shape
  n = batch * seq_len
  rows_per_w = n // NW
  nchunk = rows_per_w // CHUNK

  ids3 = input_ids.reshape(NW, nchunk, CHUNK).astype(jnp.int32)

  mesh = plsc.VectorSubcoreMesh(core_axis_name="c", subcore_axis_name="s")
  fn = pl.kernel(
      functools.partial(_body, nchunk, rows_per_w),
      out_type=jax.ShapeDtypeStruct((n, d), jnp.float32),
      mesh=mesh,
      compiler_params=pltpu.CompilerParams(use_tc_tiling_on_sc=False),
      scratch_types=[
          pltpu.VMEM((nchunk, CHUNK), jnp.int32),
          [pltpu.VMEM((CHUNK, d), jnp.float32) for _ in range(NBUF)],
          [pltpu.VMEM((CHUNK, d), jnp.float32) for _ in range(NBUF)],
          [pltpu.SemaphoreType.DMA for _ in range(NBUF)],
          [pltpu.SemaphoreType.DMA for _ in range(NBUF)],
      ],
  )
  out = fn(ids3, table)
  return out.reshape(batch, seq_len, d)
